# baseline (speedup 1.0000x reference)
import functools

import jax
import jax.numpy as jnp
from jax import lax
from jax.experimental import pallas as pl
from jax.experimental.pallas import tpu as pltpu
from jax.experimental.pallas import tpu_sc as plsc

NUM_FIELDS = 26
NUM_BUCKETS = 100000
EMBSIZE = 32
BATCH = 16384

_INFO = plsc.get_sparse_core_info()
NC, NS = _INFO.num_cores, _INFO.num_subcores
NW = NC * NS
TOTAL_ROWS = BATCH * NUM_FIELDS
ROWS_PER_W = TOTAL_ROWS // NW
GRP = 128
GROUPS_PER_W = ROWS_PER_W // GRP
GRP_PER_CHUNK = 8
CHUNK = GRP * GRP_PER_CHUNK
NCHUNKS = GROUPS_PER_W // GRP_PER_CHUNK


def _body(table_hbm, idx_hbm, out_hbm, idx_v, rows_v, sem):
    wid = lax.axis_index("s") * NC + lax.axis_index("c")
    pltpu.sync_copy(idx_hbm.at[wid], idx_v)

    def chunk_step(c, _):
        copies = [
            pltpu.async_copy(
                table_hbm.at[idx_v.at[c * GRP_PER_CHUNK + g]],
                rows_v.at[pl.ds(g * GRP, GRP)],
                sem,
            )
            for g in range(GRP_PER_CHUNK)
        ]
        for cp in copies:
            cp.wait()
        pltpu.sync_copy(rows_v, out_hbm.at[wid, c])
        return 0

    lax.fori_loop(0, NCHUNKS, chunk_step, 0)


@jax.jit
def kernel(control_inputs, tables):
    flat_table = tables.reshape(NUM_FIELDS * NUM_BUCKETS, EMBSIZE)
    offsets = (jnp.arange(NUM_FIELDS, dtype=jnp.int32) * NUM_BUCKETS)[None, :]
    gidx = (control_inputs + offsets).reshape(NW, GROUPS_PER_W, GRP)

    mesh = plsc.VectorSubcoreMesh(core_axis_name="c", subcore_axis_name="s")
    out = pl.kernel(
        _body,
        mesh=mesh,
        out_type=jax.ShapeDtypeStruct((NW, NCHUNKS, CHUNK, EMBSIZE), jnp.float32),
        scratch_types=[
            pltpu.VMEM((GROUPS_PER_W, GRP), jnp.int32),
            pltpu.VMEM((CHUNK, EMBSIZE), jnp.float32),
            pltpu.SemaphoreType.DMA,
        ],
        compiler_params=pltpu.CompilerParams(use_tc_tiling_on_sc=False),
    )(flat_table, gidx)
    return out.reshape(BATCH, NUM_FIELDS * EMBSIZE)


# speedup vs baseline: 3.3305x; 3.3305x over previous
import jax
import jax.numpy as jnp
from jax import lax
from jax.experimental import pallas as pl
from jax.experimental.pallas import tpu as pltpu
from jax.experimental.pallas import tpu_sc as plsc

NUM_FIELDS = 26
NUM_BUCKETS = 100000
EMBSIZE = 32
BATCH = 16384

_INFO = plsc.get_sparse_core_info()
NC, NS, NL = _INFO.num_cores, _INFO.num_subcores, _INFO.num_lanes
NW = NC * NS
NROWS = NUM_FIELDS * EMBSIZE
RPW = NROWS // NW
OCHUNK = 4096
NOC = BATCH // OCHUNK
VPC = OCHUNK // NL


def _body(tab_hbm, idx_hbm, out_hbm, row_v, idx_v, ob_v, gsem, wsem):
    wid = lax.axis_index("s") * NC + lax.axis_index("c")
    r0 = wid * RPW

    def row_step(k, f_prev):
        r = r0 + k
        f = r // EMBSIZE
        e = r % EMBSIZE

        @pl.when(jnp.logical_or(k == 0, f != f_prev))
        def _():
            pltpu.sync_copy(idx_hbm.at[pl.ds(f, 1)], idx_v)

        pltpu.sync_copy(tab_hbm.at[f, pl.ds(e, 1)], row_v)

        def chunk_step(q, _):
            s = q % 2

            def gvec(i, _):
                idx16 = idx_v[0, pl.ds(q * OCHUNK + i * NL, NL)]
                ob_v.at[s, 0][pl.ds(i * NL, NL)] = plsc.load_gather(row_v, [jnp.zeros((16,), jnp.int32), idx16])
                return 0

            lax.fori_loop(0, VPC, gvec, 0)
            @pl.when(q >= 2)
            def _():
                pltpu.make_async_copy(
                    ob_v.at[s], out_hbm.at[pl.ds(r, 1), pl.ds((q - 2) * OCHUNK, OCHUNK)], wsem
                ).wait()

            pltpu.async_copy(
                ob_v.at[s], out_hbm.at[pl.ds(r, 1), pl.ds(q * OCHUNK, OCHUNK)], wsem
            )
            return 0

        lax.fori_loop(0, NOC, chunk_step, 0)
        for s, q in ((NOC % 2, NOC - 2), ((NOC - 1) % 2, NOC - 1)):
            pltpu.make_async_copy(
                ob_v.at[s], out_hbm.at[pl.ds(r, 1), pl.ds(q * OCHUNK, OCHUNK)], wsem
            ).wait()
        return f

    lax.fori_loop(0, RPW, row_step, -1)


@jax.jit
def kernel(control_inputs, tables):
    tab_t = jnp.transpose(tables, (0, 2, 1))
    idx_t = jnp.transpose(control_inputs, (1, 0))

    mesh = plsc.VectorSubcoreMesh(core_axis_name="c", subcore_axis_name="s")
    out = pl.kernel(
        _body,
        mesh=mesh,
        out_type=jax.ShapeDtypeStruct((NROWS, BATCH), jnp.float32),
        scratch_types=[
            pltpu.VMEM((1, NUM_BUCKETS), jnp.float32),
            pltpu.VMEM((1, BATCH), jnp.int32),
            pltpu.VMEM((2, 1, OCHUNK), jnp.float32),
            pltpu.SemaphoreType.DMA,
            pltpu.SemaphoreType.DMA,
        ],
        compiler_params=pltpu.CompilerParams(
            use_tc_tiling_on_sc=True, needs_layout_passes=False
        ),
    )(tab_t, idx_t)
    return jnp.transpose(out, (1, 0)).reshape(BATCH, NUM_FIELDS * EMBSIZE)


# speedup vs baseline: 6.9659x; 2.0915x over previous
import jax
import jax.numpy as jnp
from jax import lax
from jax.experimental import pallas as pl
from jax.experimental.pallas import tpu as pltpu
from jax.experimental.pallas import tpu_sc as plsc

NUM_FIELDS = 26
NUM_BUCKETS = 100000
EMBSIZE = 32
BATCH = 16384

_INFO = plsc.get_sparse_core_info()
NC, NS, NL = _INFO.num_cores, _INFO.num_subcores, _INFO.num_lanes
NW = NC * NS
NROWS = NUM_FIELDS * EMBSIZE
RPW = NROWS // NW
OCHUNK = 4096
NOC = BATCH // OCHUNK
VPC = OCHUNK // NL


def _body(tab_hbm, idx_hbm, out_hbm, row_v, idx_v, ob_v, gsem, wsem):
    wid = lax.axis_index("s") * NC + lax.axis_index("c")
    r0 = wid * RPW

    def row_step(k, f_prev):
        r = r0 + k
        f = r // EMBSIZE
        e = r % EMBSIZE

        @pl.when(jnp.logical_or(k == 0, f != f_prev))
        def _():
            pltpu.sync_copy(idx_hbm.at[pl.ds(f, 1)], idx_v)

        pltpu.sync_copy(tab_hbm.at[f, pl.ds(e, 1)], row_v)

        def chunk_step(q, _):
            s = q % 2

            zero16 = jnp.zeros((NL,), jnp.int32)

            @plsc.parallel_loop(0, VPC, 1, unroll=8)
            def gvec(i):
                idx16 = idx_v[0, pl.ds(q * OCHUNK + i * NL, NL)]
                ob_v.at[s, 0][pl.ds(i * NL, NL)] = plsc.load_gather(
                    row_v, [zero16, idx16]
                )
            @pl.when(q >= 2)
            def _():
                pltpu.make_async_copy(
                    ob_v.at[s], out_hbm.at[pl.ds(r, 1), pl.ds((q - 2) * OCHUNK, OCHUNK)], wsem
                ).wait()

            pltpu.async_copy(
                ob_v.at[s], out_hbm.at[pl.ds(r, 1), pl.ds(q * OCHUNK, OCHUNK)], wsem
            )
            return 0

        lax.fori_loop(0, NOC, chunk_step, 0)
        for s, q in ((NOC % 2, NOC - 2), ((NOC - 1) % 2, NOC - 1)):
            pltpu.make_async_copy(
                ob_v.at[s], out_hbm.at[pl.ds(r, 1), pl.ds(q * OCHUNK, OCHUNK)], wsem
            ).wait()
        return f

    lax.fori_loop(0, RPW, row_step, -1)


@jax.jit
def kernel(control_inputs, tables):
    tab_t = jnp.transpose(tables, (0, 2, 1))
    idx_t = jnp.transpose(control_inputs, (1, 0))

    mesh = plsc.VectorSubcoreMesh(core_axis_name="c", subcore_axis_name="s")
    out = pl.kernel(
        _body,
        mesh=mesh,
        out_type=jax.ShapeDtypeStruct((NROWS, BATCH), jnp.float32),
        scratch_types=[
            pltpu.VMEM((1, NUM_BUCKETS), jnp.float32),
            pltpu.VMEM((1, BATCH), jnp.int32),
            pltpu.VMEM((2, 1, OCHUNK), jnp.float32),
            pltpu.SemaphoreType.DMA,
            pltpu.SemaphoreType.DMA,
        ],
        compiler_params=pltpu.CompilerParams(
            use_tc_tiling_on_sc=True, needs_layout_passes=False
        ),
    )(tab_t, idx_t)
    return jnp.transpose(out, (1, 0)).reshape(BATCH, NUM_FIELDS * EMBSIZE)
